# baseline (speedup 1.0000x reference)
import functools
import math

import jax
import jax.numpy as jnp
from jax.experimental import pallas as pl
from jax.experimental.pallas import tpu as pltpu

_BN_EPS = 1e-5
_GROUP = 8
_TILE_ROWS = 512
_VMEM_LIMIT = 48 * 1024 * 1024


def _ceil_to(x, m):
    return -(-x // m) * m


def _pass1_kernel(x_ref, w_ref, b_ref, y_ref, part_ref, *,
                  batch, tile_rows, genes, need_mask):
    i = pl.program_id(0)
    lin = jnp.dot(x_ref[...], w_ref[...],
                  preferred_element_type=jnp.float32) + b_ref[...]
    y = jnp.tanh(lin)
    y_ref[...] = y
    if need_mask:
        row = jax.lax.broadcasted_iota(jnp.int32, y.shape, 0) + i * tile_rows
        lane = jax.lax.broadcasted_iota(jnp.int32, y.shape, 1)
        b_idx = row * _GROUP + lane // genes
        y = jnp.where(b_idx < batch, y, 0.0)
    s = jnp.sum(y)
    ss = jnp.sum(y * y)
    lane3 = jax.lax.broadcasted_iota(jnp.int32, part_ref.shape, 2)
    part_ref[...] = jnp.where(lane3 == 0, s, jnp.where(lane3 == 1, ss, 0.0))


def _pass2_kernel(part_ref, y_ref, o_ref, *, inv_n):
    p = part_ref[...]
    lane = jax.lax.broadcasted_iota(jnp.int32, p.shape, 2)
    total = jnp.sum(jnp.where(lane == 0, p, 0.0))
    total_sq = jnp.sum(jnp.where(lane == 1, p, 0.0))
    mean = total * inv_n
    var = jnp.maximum(total_sq * inv_n - mean * mean, 0.0)
    inv_std = jax.lax.rsqrt(var + jnp.float32(_BN_EPS))
    o_ref[...] = (y_ref[...] - mean) * inv_std


def kernel(x, weight, bias):
    B, G, F = x.shape
    GF = G * F
    L = _GROUP * G
    K = _GROUP * GF

    x = x.astype(jnp.float32)
    weight = weight.astype(jnp.float32)
    bias = bias.astype(jnp.float32)

    B2 = _ceil_to(B, _GROUP) // _GROUP
    TR = min(_TILE_ROWS, _ceil_to(B2, 8))
    B2p = _ceil_to(B2, TR)
    nt = B2p // TR
    Bp = B2p * _GROUP

    x_flat = x.reshape(B, GF)
    if Bp != B:
        x_flat = jnp.pad(x_flat, ((0, Bp - B), (0, 0)))
    x2 = x_flat.reshape(B2p, K)

    w_bd = (weight[:, :, None] * jnp.eye(G, dtype=jnp.float32)[:, None, :]
            ).reshape(GF, G)
    w2 = jnp.kron(jnp.eye(_GROUP, dtype=jnp.float32), w_bd)
    bias2 = jnp.tile(bias, _GROUP).reshape(1, L)

    cparams = pltpu.CompilerParams(
        dimension_semantics=("parallel",),
        vmem_limit_bytes=_VMEM_LIMIT,
    )

    k1 = functools.partial(_pass1_kernel, batch=B, tile_rows=TR, genes=G,
                           need_mask=(Bp != B))
    y2, partials = pl.pallas_call(
        k1,
        out_shape=(jax.ShapeDtypeStruct((B2p, L), jnp.float32),
                   jax.ShapeDtypeStruct((nt, 1, L), jnp.float32)),
        grid=(nt,),
        in_specs=[
            pl.BlockSpec((TR, K), lambda i: (i, 0)),
            pl.BlockSpec((K, L), lambda i: (0, 0)),
            pl.BlockSpec((1, L), lambda i: (0, 0)),
        ],
        out_specs=(pl.BlockSpec((TR, L), lambda i: (i, 0)),
                   pl.BlockSpec((1, 1, L), lambda i: (i, 0, 0))),
        compiler_params=cparams,
    )(x2, w2, bias2)

    nt2 = 2 if B2p % 2 == 0 else 1
    TR2 = B2p // nt2
    k2 = functools.partial(_pass2_kernel, inv_n=1.0 / float(B * G))
    y_norm = pl.pallas_call(
        k2,
        out_shape=jax.ShapeDtypeStruct((B2p, L), jnp.float32),
        grid=(nt2,),
        in_specs=[
            pl.BlockSpec((nt, 1, L), lambda i: (0, 0, 0)),
            pl.BlockSpec((TR2, L), lambda i: (i, 0)),
        ],
        out_specs=pl.BlockSpec((TR2, L), lambda i: (i, 0)),
        compiler_params=cparams,
    )(partials, y2)

    return y_norm.reshape(Bp, G)[:B].reshape(B, G, 1)


# speedup vs baseline: 2.4981x; 2.4981x over previous
import functools

import jax
import jax.numpy as jnp
from jax.experimental import pallas as pl
from jax.experimental.pallas import tpu as pltpu

_BN_EPS = 1e-5
_TILE = 4096
_VMEM_LIMIT = 48 * 1024 * 1024


def _pass1_kernel(x_ref, w_ref, b_ref, y_ref, part_ref):
    lin = jnp.dot(x_ref[...], w_ref[...],
                  preferred_element_type=jnp.float32) + b_ref[...]
    y = jnp.tanh(lin)
    y_ref[...] = y
    s = jnp.sum(y)
    ss = jnp.sum(y * y)
    lane3 = jax.lax.broadcasted_iota(jnp.int32, part_ref.shape, 2)
    part_ref[...] = jnp.where(lane3 == 0, s, jnp.where(lane3 == 1, ss, 0.0))


def _pass2_kernel(part_ref, y_ref, o_ref, *, inv_n):
    p = part_ref[...]
    lane = jax.lax.broadcasted_iota(jnp.int32, p.shape, 2)
    total = jnp.sum(jnp.where(lane == 0, p, 0.0))
    total_sq = jnp.sum(jnp.where(lane == 1, p, 0.0))
    mean = total * inv_n
    var = jnp.maximum(total_sq * inv_n - mean * mean, 0.0)
    inv_std = jax.lax.rsqrt(var + jnp.float32(_BN_EPS))
    o_ref[...] = (y_ref[...] - mean) * inv_std


def kernel(x, weight, bias):
    B, G, F = x.shape
    GF = G * F
    x = x.astype(jnp.float32)
    weight = weight.astype(jnp.float32)
    bias = bias.astype(jnp.float32).reshape(1, G)

    TILE = min(_TILE, B)
    nt = B // TILE
    x_flat = x.reshape(B, GF)
    w_bd = (weight[:, :, None] * jnp.eye(G, dtype=jnp.float32)[:, None, :]
            ).reshape(GF, G)

    cparams = pltpu.CompilerParams(
        dimension_semantics=("parallel",),
        vmem_limit_bytes=_VMEM_LIMIT,
    )

    y, partials = pl.pallas_call(
        _pass1_kernel,
        out_shape=(jax.ShapeDtypeStruct((B, G), jnp.float32),
                   jax.ShapeDtypeStruct((nt, 1, 128), jnp.float32)),
        grid=(nt,),
        in_specs=[
            pl.BlockSpec((TILE, GF), lambda i: (i, 0)),
            pl.BlockSpec((GF, G), lambda i: (0, 0)),
            pl.BlockSpec((1, G), lambda i: (0, 0)),
        ],
        out_specs=(pl.BlockSpec((TILE, G), lambda i: (i, 0)),
                   pl.BlockSpec((1, 1, 128), lambda i: (i, 0, 0))),
        compiler_params=cparams,
    )(x_flat, w_bd, bias)

    R = (B * G) // 128
    nt2 = 2 if R % 2 == 0 else 1
    TR2 = R // nt2
    k2 = functools.partial(_pass2_kernel, inv_n=1.0 / float(B * G))
    y_norm = pl.pallas_call(
        k2,
        out_shape=jax.ShapeDtypeStruct((R, 128), jnp.float32),
        grid=(nt2,),
        in_specs=[
            pl.BlockSpec((nt, 1, 128), lambda i: (0, 0, 0)),
            pl.BlockSpec((TR2, 128), lambda i: (i, 0)),
        ],
        out_specs=pl.BlockSpec((TR2, 128), lambda i: (i, 0)),
        compiler_params=cparams,
    )(partials, y.reshape(R, 128))

    return y_norm.reshape(B, G, 1)
